# baseline (speedup 1.0000x reference)
import functools

import jax
import jax.numpy as jnp
from jax import lax
from jax.experimental import pallas as pl
from jax.experimental.pallas import tpu as pltpu
from jax.experimental.pallas import tpu_sc as plsc

VOCAB = 1000
CPAD = 1024
NPC = CPAD // 128
N_TOK = 1024 * 200
NW = 32
NSL = 2
SLICE = N_TOK // NSL
PER_W = SLICE // NW
CH = 16
PPC = CH * NPC
NCH = PER_W // CH


def _logz_body(table_ref, out_ref):
    t = table_ref[...]
    m = jnp.max(t, axis=1, keepdims=True)
    s = jnp.sum(jnp.exp(t - m), axis=1, keepdims=True)
    out_ref[...] = m + jnp.log(s)


def _logz(table):
    return pl.pallas_call(
        _logz_body,
        out_shape=jax.ShapeDtypeStruct((VOCAB, 1), jnp.float32),
    )(table)


TBLK = 512


def _tp_body(src_ref, dst_ref):
    dst_ref[...] = jnp.transpose(src_ref[...], (1, 0))


def _tp_body2(src_ref, buf_ref, dst_ref):
    del buf_ref
    dst_ref[...] = jnp.transpose(src_ref[...], (1, 0))


def _transpose_into(k, buf, lk):
    nblk = SLICE // TBLK
    base = k * nblk
    if buf is None:
        return pl.pallas_call(
            _tp_body,
            grid=(nblk,),
            in_specs=[pl.BlockSpec((TBLK, VOCAB), lambda j: (j, 0))],
            out_specs=pl.BlockSpec(
                (VOCAB, TBLK), lambda j, base=base: (0, base + j)
            ),
            out_shape=jax.ShapeDtypeStruct((VOCAB, N_TOK), jnp.float32),
        )(lk)
    return pl.pallas_call(
        _tp_body2,
        grid=(nblk,),
        in_specs=[
            pl.BlockSpec((TBLK, VOCAB), lambda j: (j, 0)),
            pl.BlockSpec(memory_space=pl.ANY),
        ],
        out_specs=pl.BlockSpec(
            (VOCAB, TBLK), lambda j, base=base: (0, base + j)
        ),
        out_shape=jax.ShapeDtypeStruct((VOCAB, N_TOK), jnp.float32),
        input_output_aliases={1: 0},
    )(lk, buf)


def _finish_body(part_ref, out_ref):
    out_ref[...] = jnp.sum(part_ref[...]).reshape(1, 1) * (1.0 / N_TOK)


def _finish(partials):
    return pl.pallas_call(
        _finish_body,
        out_shape=jax.ShapeDtypeStruct((1, 1), jnp.float32),
    )(partials)


def _sc_body(idx_hbm, tgt_hbm, tp_hbm, logz_hbm, out_hbm, part_hbm,
             idx_v, tgt_v, logz_v, acc_v,
             stag0, stag1, stag2, stag3, srow0, srow1,
             gsem0, gsem1, gsem2, gsem3, ssem0, ssem1):
    stag = (stag0, stag1, stag2, stag3)
    srow = (srow0, srow1)
    gsem = (gsem0, gsem1, gsem2, gsem3)
    ssem = (ssem0, ssem1)
    wid = lax.axis_index("s") * 2 + lax.axis_index("c")
    tbase = wid * PER_W

    pltpu.sync_copy(idx_hbm.at[pl.ds(tbase, PER_W)], idx_v)
    pltpu.sync_copy(tgt_hbm.at[pl.ds(tbase, PER_W)], tgt_v)
    pltpu.sync_copy(logz_hbm, logz_v)

    lane = lax.iota(jnp.int32, 16)

    def start_gather(g, b):
        pltpu.async_copy(
            tp_hbm.at[idx_v.at[pl.ds(g * CH, CH)]], stag[b], gsem[b]
        )

    def wait_gather(g, b):
        pltpu.make_async_copy(
            tp_hbm.at[idx_v.at[pl.ds(g * CH, CH)]], stag[b], gsem[b]
        ).wait()

    def start_scatter(g, b):
        pltpu.async_copy(srow[b], out_hbm.at[pl.ds(tbase + g * CH, CH)], ssem[b])

    def wait_scatter(g, b):
        pltpu.make_async_copy(
            srow[b], out_hbm.at[pl.ds(tbase + g * CH, CH)], ssem[b]
        ).wait()

    def retile(d, bg, sr):
        del d
        for r in range(CH):
            for ct in range(NPC):
                nfull = 8 if ct < NPC - 1 else 6
                for cc in range(nfull):
                    srow[sr][r, pl.ds(ct * 128 + cc * 16, 16)] = (
                        stag[bg][r, ct, pl.ds(cc * 16, 16)]
                    )
            tail = stag[bg][r, NPC - 1, pl.ds(96, 16)]
            plsc.store_scatter(
                srow[sr],
                [jnp.full((16,), r, jnp.int32), 992 + lane],
                tail,
                mask=lane < 8,
            )

    def loss_math(g, b, acc):
        t16 = tgt_v[pl.ds(g * CH, 16)]
        i16 = idx_v[pl.ds(g * CH, 16)]
        tl = plsc.load_gather(stag[b], [lane, t16 >> 7, t16 & 127])
        lz = plsc.load_gather(logz_v, [i16])
        return acc + (lz - tl)

    start_gather(0, 0)
    start_gather(1, 1)

    def body(g0, acc):
        for bb in range(4):
            d = g0 * 4 + bb
            bg = bb
            sg = (bb + 2) % 4
            sr = bb % 2
            @pl.when(d + 2 < NCH)
            def _():
                start_gather(d + 2, sg)
            wait_gather(d, bg)
            @pl.when(d >= 2)
            def _():
                wait_scatter(d - 2, sr)
            retile(d, bg, sr)
            start_scatter(d, sr)
            acc = loss_math(d, bg, acc)
        return acc

    acc = lax.fori_loop(0, NCH // 4, body, jnp.zeros((16,), jnp.float32))
    for h in range(NCH - 2, NCH):
        wait_scatter(h, h % 2)
    acc_v[...] = acc
    pltpu.sync_copy(acc_v, part_hbm.at[wid])


@functools.partial(jax.jit, donate_argnums=())
def _sc_gather(idx_flat, tgt_flat, tpieces, logz):
    mesh = plsc.VectorSubcoreMesh(core_axis_name="c", subcore_axis_name="s")
    f = functools.partial(
        pl.kernel,
        mesh=mesh,
        compiler_params=pltpu.CompilerParams(
            use_tc_tiling_on_sc=True, needs_layout_passes=False
        ),
        out_type=[
            jax.ShapeDtypeStruct((SLICE, VOCAB), jnp.float32),
            jax.ShapeDtypeStruct((NW, 16), jnp.float32),
        ],
        scratch_types=[
            pltpu.VMEM((PER_W,), jnp.int32),
            pltpu.VMEM((PER_W,), jnp.int32),
            pltpu.VMEM((VOCAB,), jnp.float32),
            pltpu.VMEM((16,), jnp.float32),
            pltpu.VMEM((CH, NPC, 128), jnp.float32),
            pltpu.VMEM((CH, NPC, 128), jnp.float32),
            pltpu.VMEM((CH, NPC, 128), jnp.float32),
            pltpu.VMEM((CH, NPC, 128), jnp.float32),
            pltpu.VMEM((CH, VOCAB), jnp.float32),
            pltpu.VMEM((CH, VOCAB), jnp.float32),
            pltpu.SemaphoreType.DMA,
            pltpu.SemaphoreType.DMA,
            pltpu.SemaphoreType.DMA,
            pltpu.SemaphoreType.DMA,
            pltpu.SemaphoreType.DMA,
            pltpu.SemaphoreType.DMA,
        ],
    )(_sc_body)
    return f(idx_flat, tgt_flat, tpieces, logz)


def kernel(idx, targets, token_embedding_table):
    idx_flat = idx.reshape(-1).astype(jnp.int32)
    tgt_flat = targets.reshape(-1).astype(jnp.int32)
    table = token_embedding_table.astype(jnp.float32)
    tpieces = jnp.pad(table, ((0, 0), (0, CPAD - VOCAB))).reshape(
        VOCAB, NPC, 128
    )
    logz = _logz(table).reshape(-1)
    buf = None
    parts = []
    for k in range(NSL):
        lk, pk = _sc_gather(
            lax.slice(idx_flat, (k * SLICE,), ((k + 1) * SLICE,)),
            lax.slice(tgt_flat, (k * SLICE,), ((k + 1) * SLICE,)),
            tpieces, logz,
        )
        buf = _transpose_into(k, buf, lk)
        parts.append(pk)
    logits = buf.T
    loss = _finish(jnp.concatenate(parts, axis=0))[0, 0]
    return (logits, loss)


# speedup vs baseline: 1.0250x; 1.0250x over previous
import functools

import jax
import jax.numpy as jnp
from jax import lax
from jax.experimental import pallas as pl
from jax.experimental.pallas import tpu as pltpu
from jax.experimental.pallas import tpu_sc as plsc

VOCAB = 1000
CPAD = 1024
NPC = CPAD // 128
N_TOK = 1024 * 200
NW = 32
PER_W = N_TOK // NW
CH = 16
PPC = CH * NPC
NCH = PER_W // CH


def _logz_body(table_ref, out_ref):
    t = table_ref[...]
    m = jnp.max(t, axis=1, keepdims=True)
    s = jnp.sum(jnp.exp(t - m), axis=1, keepdims=True)
    out_ref[...] = m + jnp.log(s)


def _logz(table):
    return pl.pallas_call(
        _logz_body,
        out_shape=jax.ShapeDtypeStruct((VOCAB, 1), jnp.float32),
    )(table)


TBLK = 512


def _tp_body(src_ref, dst_ref):
    dst_ref[...] = jnp.transpose(src_ref[...], (1, 0))


def _transpose_all(lk):
    nblk = N_TOK // TBLK
    return pl.pallas_call(
        _tp_body,
        grid=(nblk,),
        in_specs=[pl.BlockSpec((TBLK, VOCAB), lambda j: (j, 0))],
        out_specs=pl.BlockSpec((VOCAB, TBLK), lambda j: (0, j)),
        out_shape=jax.ShapeDtypeStruct((VOCAB, N_TOK), jnp.float32),
    )(lk)


def _finish_body(part_ref, out_ref):
    out_ref[...] = jnp.sum(part_ref[...]).reshape(1, 1) * (1.0 / N_TOK)


def _finish(partials):
    return pl.pallas_call(
        _finish_body,
        out_shape=jax.ShapeDtypeStruct((1, 1), jnp.float32),
    )(partials)


def _sc_body(idx_hbm, tgt_hbm, tp_hbm, logz_hbm, out_hbm, part_hbm,
             idx_v, tgt_v, logz_v, acc_v,
             stag0, stag1, stag2, stag3, srow0, srow1,
             gsem0, gsem1, gsem2, gsem3, ssem0, ssem1):
    stag = (stag0, stag1, stag2, stag3)
    srow = (srow0, srow1)
    gsem = (gsem0, gsem1, gsem2, gsem3)
    ssem = (ssem0, ssem1)
    wid = lax.axis_index("s") * 2 + lax.axis_index("c")
    tbase = wid * PER_W

    pltpu.sync_copy(idx_hbm.at[pl.ds(tbase, PER_W)], idx_v)
    pltpu.sync_copy(tgt_hbm.at[pl.ds(tbase, PER_W)], tgt_v)
    pltpu.sync_copy(logz_hbm, logz_v)

    lane = lax.iota(jnp.int32, 16)

    def start_gather(g, b):
        pltpu.async_copy(
            tp_hbm.at[idx_v.at[pl.ds(g * CH, CH)]], stag[b], gsem[b]
        )

    def wait_gather(g, b):
        pltpu.make_async_copy(
            tp_hbm.at[idx_v.at[pl.ds(g * CH, CH)]], stag[b], gsem[b]
        ).wait()

    def start_scatter(g, b):
        pltpu.async_copy(srow[b], out_hbm.at[pl.ds(tbase + g * CH, CH)], ssem[b])

    def wait_scatter(g, b):
        pltpu.make_async_copy(
            srow[b], out_hbm.at[pl.ds(tbase + g * CH, CH)], ssem[b]
        ).wait()

    def retile(d, bg, sr):
        del d
        for r in range(CH):
            for ct in range(NPC):
                nfull = 8 if ct < NPC - 1 else 6
                for cc in range(nfull):
                    srow[sr][r, pl.ds(ct * 128 + cc * 16, 16)] = (
                        stag[bg][r, ct, pl.ds(cc * 16, 16)]
                    )
            tail = stag[bg][r, NPC - 1, pl.ds(96, 16)]
            plsc.store_scatter(
                srow[sr],
                [jnp.full((16,), r, jnp.int32), 992 + lane],
                tail,
                mask=lane < 8,
            )

    def loss_math(g, b, acc):
        t16 = tgt_v[pl.ds(g * CH, 16)]
        i16 = idx_v[pl.ds(g * CH, 16)]
        tl = plsc.load_gather(stag[b], [lane, t16 >> 7, t16 & 127])
        lz = plsc.load_gather(logz_v, [i16])
        return acc + (lz - tl)

    start_gather(0, 0)
    start_gather(1, 1)

    def body(g0, acc):
        for bb in range(4):
            d = g0 * 4 + bb
            bg = bb
            sg = (bb + 2) % 4
            sr = bb % 2
            @pl.when(d + 2 < NCH)
            def _():
                start_gather(d + 2, sg)
            wait_gather(d, bg)
            @pl.when(d >= 2)
            def _():
                wait_scatter(d - 2, sr)
            retile(d, bg, sr)
            start_scatter(d, sr)
            acc = loss_math(d, bg, acc)
        return acc

    acc = lax.fori_loop(0, NCH // 4, body, jnp.zeros((16,), jnp.float32))
    for h in range(NCH - 2, NCH):
        wait_scatter(h, h % 2)
    acc_v[...] = acc
    pltpu.sync_copy(acc_v, part_hbm.at[wid])


@functools.partial(jax.jit, donate_argnums=())
def _sc_gather(idx_flat, tgt_flat, tpieces, logz):
    mesh = plsc.VectorSubcoreMesh(core_axis_name="c", subcore_axis_name="s")
    f = functools.partial(
        pl.kernel,
        mesh=mesh,
        compiler_params=pltpu.CompilerParams(
            use_tc_tiling_on_sc=True, needs_layout_passes=False
        ),
        out_type=[
            jax.ShapeDtypeStruct((N_TOK, VOCAB), jnp.float32),
            jax.ShapeDtypeStruct((NW, 16), jnp.float32),
        ],
        scratch_types=[
            pltpu.VMEM((PER_W,), jnp.int32),
            pltpu.VMEM((PER_W,), jnp.int32),
            pltpu.VMEM((VOCAB,), jnp.float32),
            pltpu.VMEM((16,), jnp.float32),
            pltpu.VMEM((CH, NPC, 128), jnp.float32),
            pltpu.VMEM((CH, NPC, 128), jnp.float32),
            pltpu.VMEM((CH, NPC, 128), jnp.float32),
            pltpu.VMEM((CH, NPC, 128), jnp.float32),
            pltpu.VMEM((CH, VOCAB), jnp.float32),
            pltpu.VMEM((CH, VOCAB), jnp.float32),
            pltpu.SemaphoreType.DMA,
            pltpu.SemaphoreType.DMA,
            pltpu.SemaphoreType.DMA,
            pltpu.SemaphoreType.DMA,
            pltpu.SemaphoreType.DMA,
            pltpu.SemaphoreType.DMA,
        ],
    )(_sc_body)
    return f(idx_flat, tgt_flat, tpieces, logz)


def kernel(idx, targets, token_embedding_table):
    idx_flat = idx.reshape(-1).astype(jnp.int32)
    tgt_flat = targets.reshape(-1).astype(jnp.int32)
    table = token_embedding_table.astype(jnp.float32)
    tpieces = jnp.pad(table, ((0, 0), (0, CPAD - VOCAB))).reshape(
        VOCAB, NPC, 128
    )
    logz = _logz(table).reshape(-1)
    lk, partials = _sc_gather(idx_flat, tgt_flat, tpieces, logz)
    logits = _transpose_all(lk).T
    loss = _finish(partials)[0, 0]
    return (logits, loss)
